# baseline (speedup 1.0000x reference)
import functools

import jax
import jax.numpy as jnp
from jax import lax
from jax.experimental import pallas as pl
from jax.experimental.pallas import tpu as pltpu
from jax.experimental.pallas import tpu_sc as plsc

_EPS = 1e-5
_G = 8
_NC = 2
_NS = 16
_LANES = 16
_NW = _NC * _NS
_CHUNK = 80
_NBUF = 4
_ROWBLK = 1000


def _sc_degree(packed, n_pad):
    n_chunks = packed.shape[0]
    nk = n_chunks // _NW
    rpt = n_pad // _NS
    zeros_hbm = jnp.zeros((rpt,), jnp.float32)
    mesh = plsc.VectorSubcoreMesh(core_axis_name="c", subcore_axis_name="s")

    @functools.partial(
        pl.kernel, mesh=mesh,
        out_type=jax.ShapeDtypeStruct((_NC * n_pad,), jnp.float32),
        scratch_types=[
            pltpu.VMEM((2, _CHUNK), jnp.int32),
            pltpu.VMEM((_CHUNK,), jnp.float32),
            pltpu.VMEM_SHARED((n_pad,), jnp.float32),
        ],
    )
    def deg_kernel(pk_hbm, z_hbm, out_hbm, idx_v, ones_v, acc):
        c = lax.axis_index("c")
        s = lax.axis_index("s")
        wid = s * _NC + c
        pltpu.sync_copy(z_hbm, acc.at[pl.ds(s * rpt, rpt)])
        for i in range(_CHUNK // _LANES):
            ones_v[pl.ds(i * _LANES, _LANES)] = jnp.ones((_LANES,), jnp.float32)
        plsc.subcore_barrier()

        def body(k, carry):
            pltpu.sync_copy(pk_hbm.at[wid + k * _NW], idx_v)
            pltpu.sync_copy(ones_v, acc.at[idx_v.at[1]], add=True)
            return carry

        lax.fori_loop(0, nk, body, 0)
        plsc.subcore_barrier()
        pltpu.sync_copy(acc.at[pl.ds(s * rpt, rpt)],
                        out_hbm.at[pl.ds(c * n_pad + s * rpt, rpt)])

    return deg_kernel(packed, zeros_hbm)


def _sc_msgpass(p, packed, n_pad):
    n, d = p.shape
    n_chunks = packed.shape[0]
    nk = n_chunks // _NW
    rpt = n_pad // _NS
    zeros_hbm = jnp.zeros((rpt, d), jnp.float32)
    mesh = plsc.VectorSubcoreMesh(core_axis_name="c", subcore_axis_name="s")

    nbuf = _NBUF

    @functools.partial(
        pl.kernel, mesh=mesh,
        out_type=jax.ShapeDtypeStruct((_NC, n_pad, d), jnp.float32),
        scratch_types=(
            [pltpu.VMEM((2, _CHUNK), jnp.int32) for _ in range(nbuf)]
            + [pltpu.VMEM((_CHUNK, d), jnp.float32) for _ in range(nbuf)]
            + [pltpu.VMEM_SHARED((n_pad, d), jnp.float32)]
        ),
    )
    def mp_kernel(p_hbm, pk_hbm, z_hbm, out_hbm, *bufs):
        ibufs = bufs[:nbuf]
        rbufs = bufs[nbuf:2 * nbuf]
        acc = bufs[2 * nbuf]
        c = lax.axis_index("c")
        s = lax.axis_index("s")
        wid = s * _NC + c
        pltpu.sync_copy(z_hbm, acc.at[pl.ds(s * rpt, rpt)])
        plsc.subcore_barrier()

        def scoped(*sems):
            sgs = sems[:nbuf]
            sss = sems[nbuf:]
            for t in range(nbuf):
                pltpu.sync_copy(pk_hbm.at[wid + t * _NW], ibufs[t])
                pltpu.async_copy(p_hbm.at[ibufs[t].at[0]], rbufs[t], sgs[t])

            def body(j, carry):
                for t in range(nbuf):
                    k = nbuf * j + t
                    pltpu.make_async_copy(
                        p_hbm.at[ibufs[t].at[0]], rbufs[t], sgs[t]).wait()
                    pltpu.async_copy(rbufs[t], acc.at[ibufs[t].at[1]],
                                     sss[t], add=True)

                    @pl.when(k + nbuf < nk)
                    def _():
                        pltpu.make_async_copy(
                            rbufs[t], acc.at[ibufs[t].at[1]], sss[t]).wait()
                        pltpu.sync_copy(pk_hbm.at[wid + (k + nbuf) * _NW],
                                        ibufs[t])
                        pltpu.async_copy(p_hbm.at[ibufs[t].at[0]],
                                         rbufs[t], sgs[t])
                return carry

            lax.fori_loop(0, nk // nbuf, body, 0)
            for t in range(nbuf):
                pltpu.make_async_copy(
                    rbufs[t], acc.at[ibufs[t].at[1]], sss[t]).wait()

        pl.run_scoped(scoped,
                      *([pltpu.SemaphoreType.DMA] * (2 * nbuf)))
        plsc.subcore_barrier()
        pltpu.sync_copy(acc.at[pl.ds(s * rpt, rpt)],
                        out_hbm.at[c, pl.ds(s * rpt, rpt)])

    return mp_kernel(p, packed, zeros_hbm)


def _tc_prep(deg_partials, batch2d):
    n_pad = deg_partials.shape[1]

    def body(dp_ref, b_ref, dinv_ref, cnt_ref):
        deg = dp_ref[0, :] + dp_ref[1, :] + 1.0
        dinv_ref[...] = lax.rsqrt(deg)[None, :]
        iota = lax.broadcasted_iota(jnp.int32, (1, _G), 1)
        onehot = (b_ref[...] == iota).astype(jnp.float32)
        cnt = jnp.maximum(jnp.sum(onehot, axis=0), 1.0)
        cnt_ref[...] = jnp.broadcast_to(cnt[:, None], (_G, 128))

    return pl.pallas_call(
        body,
        out_shape=(
            jax.ShapeDtypeStruct((1, n_pad), jnp.float32),
            jax.ShapeDtypeStruct((_G, 128), jnp.float32),
        ),
    )(deg_partials, batch2d)


def _tc_in_proj(x, w, b2d, dinv_col):
    n, d_in = x.shape
    d_out = w.shape[1]
    grid = (n // _ROWBLK,)

    def body(x_ref, w_ref, b_ref, dv_ref, out_ref):
        h = jnp.dot(x_ref[...], w_ref[...],
                    preferred_element_type=jnp.float32) + b_ref[...]
        out_ref[...] = dv_ref[...] * h

    return pl.pallas_call(
        body,
        grid=grid,
        in_specs=[
            pl.BlockSpec((_ROWBLK, d_in), lambda i: (i, 0)),
            pl.BlockSpec((d_in, d_out), lambda i: (0, 0)),
            pl.BlockSpec((1, d_out), lambda i: (0, 0)),
            pl.BlockSpec((_ROWBLK, 1), lambda i: (i, 0)),
        ],
        out_shape=jax.ShapeDtypeStruct((n, d_out), jnp.float32),
        out_specs=pl.BlockSpec((_ROWBLK, d_out), lambda i: (i, 0)),
    )(x, w, b2d, dinv_col)


def _tc_layer_pre(qab, p, dinv_col, w, b2d, batch2d):
    n, d_in = p.shape
    d_out = w.shape[1]
    grid = (n // _ROWBLK,)

    def body(q_ref, p_ref, dv_ref, w_ref, b_ref, bt_ref,
             z_ref, s1_ref, s2_ref):
        i = pl.program_id(0)
        agg = dv_ref[...] * (q_ref[0] + q_ref[1] + p_ref[...])
        z = jnp.dot(agg, w_ref[...],
                    preferred_element_type=jnp.float32) + b_ref[...]
        z_ref[...] = z
        iota = lax.broadcasted_iota(jnp.int32, (1, _G), 1)
        onehot = (bt_ref[...] == iota).astype(jnp.float32)
        dn = (((0,), (0,)), ((), ()))
        s1 = lax.dot_general(onehot, z, dn,
                             preferred_element_type=jnp.float32)
        s2 = lax.dot_general(onehot, z * z, dn,
                             preferred_element_type=jnp.float32)

        @pl.when(i == 0)
        def _():
            s1_ref[...] = s1
            s2_ref[...] = s2

        @pl.when(i > 0)
        def _():
            s1_ref[...] += s1
            s2_ref[...] += s2

    return pl.pallas_call(
        body,
        grid=grid,
        in_specs=[
            pl.BlockSpec((2, _ROWBLK, d_in), lambda i: (0, i, 0)),
            pl.BlockSpec((_ROWBLK, d_in), lambda i: (i, 0)),
            pl.BlockSpec((_ROWBLK, 1), lambda i: (i, 0)),
            pl.BlockSpec((d_in, d_out), lambda i: (0, 0)),
            pl.BlockSpec((1, d_out), lambda i: (0, 0)),
            pl.BlockSpec((_ROWBLK, 1), lambda i: (i, 0)),
        ],
        out_shape=(
            jax.ShapeDtypeStruct((n, d_out), jnp.float32),
            jax.ShapeDtypeStruct((_G, d_out), jnp.float32),
            jax.ShapeDtypeStruct((_G, d_out), jnp.float32),
        ),
        out_specs=(
            pl.BlockSpec((_ROWBLK, d_out), lambda i: (i, 0)),
            pl.BlockSpec((_G, d_out), lambda i: (0, 0)),
            pl.BlockSpec((_G, d_out), lambda i: (0, 0)),
        ),
    )(qab, p, dinv_col, w, b2d, batch2d)


def _gnorm_rows(z, bt, s1_ref, s2_ref, cnt_ref, w_ref, bb_ref, a_ref):
    cntc = cnt_ref[:, 0:1]
    m = s1_ref[...] / cntc
    e2 = s2_ref[...] / cntc
    a = a_ref[...]
    var = e2 + (a * a - 2.0 * a) * (m * m)
    inv = lax.rsqrt(var + _EPS)
    iota = lax.broadcasted_iota(jnp.int32, (1, _G), 1)
    onehot = (bt == iota).astype(jnp.float32)
    rm = jnp.dot(onehot, m, preferred_element_type=jnp.float32)
    rinv = jnp.dot(onehot, inv, preferred_element_type=jnp.float32)
    out = (z - a * rm) * rinv
    return jnp.maximum(w_ref[...] * out + bb_ref[...], 0.0), onehot


def _tc_layer_post(z, s1, s2, cnt, batch2d, dinv_col, w2d, bb2d, a2d):
    n, d = z.shape
    grid = (n // _ROWBLK,)

    def body(z_ref, s1_ref, s2_ref, cnt_ref, bt_ref, dv_ref,
             w_ref, bb_ref, a_ref, out_ref):
        h, _ = _gnorm_rows(z_ref[...], bt_ref[...], s1_ref, s2_ref,
                           cnt_ref, w_ref, bb_ref, a_ref)
        out_ref[...] = dv_ref[...] * h

    return pl.pallas_call(
        body,
        grid=grid,
        in_specs=[
            pl.BlockSpec((_ROWBLK, d), lambda i: (i, 0)),
            pl.BlockSpec((_G, d), lambda i: (0, 0)),
            pl.BlockSpec((_G, d), lambda i: (0, 0)),
            pl.BlockSpec((_G, 128), lambda i: (0, 0)),
            pl.BlockSpec((_ROWBLK, 1), lambda i: (i, 0)),
            pl.BlockSpec((_ROWBLK, 1), lambda i: (i, 0)),
            pl.BlockSpec((1, d), lambda i: (0, 0)),
            pl.BlockSpec((1, d), lambda i: (0, 0)),
            pl.BlockSpec((1, d), lambda i: (0, 0)),
        ],
        out_shape=jax.ShapeDtypeStruct((n, d), jnp.float32),
        out_specs=pl.BlockSpec((_ROWBLK, d), lambda i: (i, 0)),
    )(z, s1, s2, cnt, batch2d, dinv_col, w2d, bb2d, a2d)


def _tc_final_pool(z, s1, s2, cnt, batch2d, w2d, bb2d, a2d):
    n, d = z.shape
    grid = (n // _ROWBLK,)
    last = n // _ROWBLK - 1

    def body(z_ref, s1_ref, s2_ref, cnt_ref, bt_ref,
             w_ref, bb_ref, a_ref, mean_ref, max_ref):
        i = pl.program_id(0)
        h, onehot = _gnorm_rows(z_ref[...], bt_ref[...], s1_ref, s2_ref,
                                cnt_ref, w_ref, bb_ref, a_ref)
        dn = (((0,), (0,)), ((), ()))
        hs = lax.dot_general(onehot, h, dn,
                             preferred_element_type=jnp.float32)
        bt = bt_ref[...]
        gmax = []
        for g in range(_G):
            mask = bt == g
            gmax.append(jnp.max(jnp.where(mask, h, -jnp.inf), axis=0))
        gm = jnp.stack(gmax, axis=0)

        @pl.when(i == 0)
        def _():
            mean_ref[...] = hs
            max_ref[...] = gm

        @pl.when(i > 0)
        def _():
            mean_ref[...] += hs
            max_ref[...] = jnp.maximum(max_ref[...], gm)

        @pl.when(i == last)
        def _():
            mean_ref[...] = mean_ref[...] / cnt_ref[:, 0:1]

    return pl.pallas_call(
        body,
        grid=grid,
        in_specs=[
            pl.BlockSpec((_ROWBLK, d), lambda i: (i, 0)),
            pl.BlockSpec((_G, d), lambda i: (0, 0)),
            pl.BlockSpec((_G, d), lambda i: (0, 0)),
            pl.BlockSpec((_G, 128), lambda i: (0, 0)),
            pl.BlockSpec((_ROWBLK, 1), lambda i: (i, 0)),
            pl.BlockSpec((1, d), lambda i: (0, 0)),
            pl.BlockSpec((1, d), lambda i: (0, 0)),
            pl.BlockSpec((1, d), lambda i: (0, 0)),
        ],
        out_shape=(
            jax.ShapeDtypeStruct((_G, d), jnp.float32),
            jax.ShapeDtypeStruct((_G, d), jnp.float32),
        ),
        out_specs=(
            pl.BlockSpec((_G, d), lambda i: (0, 0)),
            pl.BlockSpec((_G, d), lambda i: (0, 0)),
        ),
    )(z, s1, s2, cnt, batch2d, w2d, bb2d, a2d)


def kernel(x, edge_index, batch, W_in, b_in, W0, b0, gnw0, gnb0, gna0,
           W1, b1, gnw1, gnb1, gna1, W2, b2, gnw2, gnb2, gna2):
    n = x.shape[0]
    n_edges = edge_index.shape[1]
    batch2d = batch[:, None]

    n_pad = ((n + _NS * 16 - 1) // (_NS * 16)) * (_NS * 16)
    cpw = _NBUF * _NW * _CHUNK
    n_e_pad = ((n_edges + cpw - 1) // cpw) * cpw
    n_extra = n_e_pad - n_edges
    pad_src = (jnp.arange(n_extra, dtype=jnp.int32) * 7) % n
    pad_dst = n + (jnp.arange(n_extra, dtype=jnp.int32) % (n_pad - n))
    src = jnp.concatenate([edge_index[0], pad_src])
    dst = jnp.concatenate([edge_index[1], pad_dst])
    packed = jnp.stack([src, dst]).reshape(2, n_e_pad // _CHUNK,
                                           _CHUNK).transpose(1, 0, 2)

    deg_partials = _sc_degree(packed, n_pad).reshape(_NC, n_pad)
    dinv_row, cnt = _tc_prep(deg_partials, batch2d)
    dinv_col = dinv_row[0, :n, None]

    p = _tc_in_proj(x, W_in, b_in[None, :], dinv_col)
    layers = [
        (W0, b0, gnw0, gnb0, gna0),
        (W1, b1, gnw1, gnb1, gna1),
        (W2, b2, gnw2, gnb2, gna2),
    ]
    for li, (w, b, gw, gb, ga) in enumerate(layers):
        qab = _sc_msgpass(p, packed, n_pad)
        z, s1, s2 = _tc_layer_pre(qab, p, dinv_col, w, b[None, :], batch2d)
        if li < 2:
            p = _tc_layer_post(z, s1, s2, cnt, batch2d, dinv_col,
                               gw[None, :], gb[None, :], ga[None, :])
        else:
            pmean, pmax = _tc_final_pool(z, s1, s2, cnt, batch2d,
                                         gw[None, :], gb[None, :],
                                         ga[None, :])
    return jnp.concatenate([pmean, pmax], axis=1)


# speedup vs baseline: 1.1369x; 1.1369x over previous
import functools

import jax
import jax.numpy as jnp
from jax import lax
from jax.experimental import pallas as pl
from jax.experimental.pallas import tpu as pltpu
from jax.experimental.pallas import tpu_sc as plsc

_EPS = 1e-5
_G = 8
_NC = 2
_NS = 16
_LANES = 16
_NW = _NC * _NS
_CHUNK = 128
_NBUF = 2
_ROWBLK = 1000


def _sc_degree(packed, n_pad):
    n_chunks = packed.shape[0]
    nk = n_chunks // _NW
    rpt = n_pad // _NS
    zeros_hbm = jnp.zeros((rpt,), jnp.float32)
    mesh = plsc.VectorSubcoreMesh(core_axis_name="c", subcore_axis_name="s")

    @functools.partial(
        pl.kernel, mesh=mesh,
        out_type=jax.ShapeDtypeStruct((_NC * n_pad,), jnp.float32),
        scratch_types=[
            pltpu.VMEM((2, _CHUNK), jnp.int32),
            pltpu.VMEM((_CHUNK,), jnp.float32),
            pltpu.VMEM_SHARED((n_pad,), jnp.float32),
        ],
    )
    def deg_kernel(pk_hbm, z_hbm, out_hbm, idx_v, ones_v, acc):
        c = lax.axis_index("c")
        s = lax.axis_index("s")
        wid = s * _NC + c
        pltpu.sync_copy(z_hbm, acc.at[pl.ds(s * rpt, rpt)])
        for i in range(_CHUNK // _LANES):
            ones_v[pl.ds(i * _LANES, _LANES)] = jnp.ones((_LANES,), jnp.float32)
        plsc.subcore_barrier()

        def body(k, carry):
            pltpu.sync_copy(pk_hbm.at[wid + k * _NW], idx_v)
            pltpu.sync_copy(ones_v, acc.at[idx_v.at[1]], add=True)
            return carry

        lax.fori_loop(0, nk, body, 0)
        plsc.subcore_barrier()
        pltpu.sync_copy(acc.at[pl.ds(s * rpt, rpt)],
                        out_hbm.at[pl.ds(c * n_pad + s * rpt, rpt)])

    return deg_kernel(packed, zeros_hbm)


def _sc_msgpass(p, packed, n_pad):
    n, d = p.shape
    n_chunks = packed.shape[0]
    nk = n_chunks // _NW
    rpt = n_pad // _NS
    zeros_hbm = jnp.zeros((rpt, d), jnp.float32)
    mesh = plsc.VectorSubcoreMesh(core_axis_name="c", subcore_axis_name="s")

    nbuf = _NBUF

    @functools.partial(
        pl.kernel, mesh=mesh,
        out_type=jax.ShapeDtypeStruct((_NC, n_pad, d), jnp.float32),
        scratch_types=(
            [pltpu.VMEM((2, _CHUNK), jnp.int32) for _ in range(nbuf)]
            + [pltpu.VMEM((_CHUNK, d), jnp.float32) for _ in range(nbuf)]
            + [pltpu.VMEM_SHARED((n_pad, d), jnp.float32)]
        ),
    )
    def mp_kernel(p_hbm, pk_hbm, z_hbm, out_hbm, *bufs):
        ibufs = bufs[:nbuf]
        rbufs = bufs[nbuf:2 * nbuf]
        acc = bufs[2 * nbuf]
        c = lax.axis_index("c")
        s = lax.axis_index("s")
        wid = s * _NC + c
        pltpu.sync_copy(z_hbm, acc.at[pl.ds(s * rpt, rpt)])
        plsc.subcore_barrier()

        def scoped(*sems):
            sgs = sems[:nbuf]
            sss = sems[nbuf:]
            for t in range(nbuf):
                pltpu.sync_copy(pk_hbm.at[wid + t * _NW], ibufs[t])
                pltpu.async_copy(p_hbm.at[ibufs[t].at[0]], rbufs[t], sgs[t])

            def body(j, carry):
                for t in range(nbuf):
                    k = nbuf * j + t
                    pltpu.make_async_copy(
                        p_hbm.at[ibufs[t].at[0]], rbufs[t], sgs[t]).wait()
                    pltpu.async_copy(rbufs[t], acc.at[ibufs[t].at[1]],
                                     sss[t], add=True)

                    @pl.when(k + nbuf < nk)
                    def _():
                        pltpu.make_async_copy(
                            rbufs[t], acc.at[ibufs[t].at[1]], sss[t]).wait()
                        pltpu.sync_copy(pk_hbm.at[wid + (k + nbuf) * _NW],
                                        ibufs[t])
                        pltpu.async_copy(p_hbm.at[ibufs[t].at[0]],
                                         rbufs[t], sgs[t])
                return carry

            lax.fori_loop(0, nk // nbuf, body, 0)
            for t in range(nbuf):
                pltpu.make_async_copy(
                    rbufs[t], acc.at[ibufs[t].at[1]], sss[t]).wait()

        pl.run_scoped(scoped,
                      *([pltpu.SemaphoreType.DMA] * (2 * nbuf)))
        plsc.subcore_barrier()
        pltpu.sync_copy(acc.at[pl.ds(s * rpt, rpt)],
                        out_hbm.at[c, pl.ds(s * rpt, rpt)])

    return mp_kernel(p, packed, zeros_hbm)


def _tc_prep(deg_partials, batch2d):
    n_pad = deg_partials.shape[1]

    def body(dp_ref, b_ref, dinv_ref, cnt_ref):
        deg = dp_ref[0, :] + dp_ref[1, :] + 1.0
        dinv_ref[...] = lax.rsqrt(deg)[None, :]
        iota = lax.broadcasted_iota(jnp.int32, (1, _G), 1)
        onehot = (b_ref[...] == iota).astype(jnp.float32)
        cnt = jnp.maximum(jnp.sum(onehot, axis=0), 1.0)
        cnt_ref[...] = jnp.broadcast_to(cnt[:, None], (_G, 128))

    return pl.pallas_call(
        body,
        out_shape=(
            jax.ShapeDtypeStruct((1, n_pad), jnp.float32),
            jax.ShapeDtypeStruct((_G, 128), jnp.float32),
        ),
    )(deg_partials, batch2d)


def _tc_in_proj(x, w, b2d, dinv_col):
    n, d_in = x.shape
    d_out = w.shape[1]
    grid = (n // _ROWBLK,)

    def body(x_ref, w_ref, b_ref, dv_ref, out_ref):
        h = jnp.dot(x_ref[...], w_ref[...],
                    preferred_element_type=jnp.float32) + b_ref[...]
        out_ref[...] = dv_ref[...] * h

    return pl.pallas_call(
        body,
        grid=grid,
        in_specs=[
            pl.BlockSpec((_ROWBLK, d_in), lambda i: (i, 0)),
            pl.BlockSpec((d_in, d_out), lambda i: (0, 0)),
            pl.BlockSpec((1, d_out), lambda i: (0, 0)),
            pl.BlockSpec((_ROWBLK, 1), lambda i: (i, 0)),
        ],
        out_shape=jax.ShapeDtypeStruct((n, d_out), jnp.float32),
        out_specs=pl.BlockSpec((_ROWBLK, d_out), lambda i: (i, 0)),
    )(x, w, b2d, dinv_col)


def _tc_layer_pre(qab, p, dinv_col, w, b2d, batch2d):
    n, d_in = p.shape
    d_out = w.shape[1]
    grid = (n // _ROWBLK,)

    def body(q_ref, p_ref, dv_ref, w_ref, b_ref, bt_ref,
             z_ref, s1_ref, s2_ref):
        i = pl.program_id(0)
        agg = dv_ref[...] * (q_ref[0] + q_ref[1] + p_ref[...])
        z = jnp.dot(agg, w_ref[...],
                    preferred_element_type=jnp.float32) + b_ref[...]
        z_ref[...] = z
        iota = lax.broadcasted_iota(jnp.int32, (1, _G), 1)
        onehot = (bt_ref[...] == iota).astype(jnp.float32)
        dn = (((0,), (0,)), ((), ()))
        s1 = lax.dot_general(onehot, z, dn,
                             preferred_element_type=jnp.float32)
        s2 = lax.dot_general(onehot, z * z, dn,
                             preferred_element_type=jnp.float32)

        @pl.when(i == 0)
        def _():
            s1_ref[...] = s1
            s2_ref[...] = s2

        @pl.when(i > 0)
        def _():
            s1_ref[...] += s1
            s2_ref[...] += s2

    return pl.pallas_call(
        body,
        grid=grid,
        in_specs=[
            pl.BlockSpec((2, _ROWBLK, d_in), lambda i: (0, i, 0)),
            pl.BlockSpec((_ROWBLK, d_in), lambda i: (i, 0)),
            pl.BlockSpec((_ROWBLK, 1), lambda i: (i, 0)),
            pl.BlockSpec((d_in, d_out), lambda i: (0, 0)),
            pl.BlockSpec((1, d_out), lambda i: (0, 0)),
            pl.BlockSpec((_ROWBLK, 1), lambda i: (i, 0)),
        ],
        out_shape=(
            jax.ShapeDtypeStruct((n, d_out), jnp.float32),
            jax.ShapeDtypeStruct((_G, d_out), jnp.float32),
            jax.ShapeDtypeStruct((_G, d_out), jnp.float32),
        ),
        out_specs=(
            pl.BlockSpec((_ROWBLK, d_out), lambda i: (i, 0)),
            pl.BlockSpec((_G, d_out), lambda i: (0, 0)),
            pl.BlockSpec((_G, d_out), lambda i: (0, 0)),
        ),
    )(qab, p, dinv_col, w, b2d, batch2d)


def _gnorm_rows(z, bt, s1_ref, s2_ref, cnt_ref, w_ref, bb_ref, a_ref):
    cntc = cnt_ref[:, 0:1]
    m = s1_ref[...] / cntc
    e2 = s2_ref[...] / cntc
    a = a_ref[...]
    var = e2 + (a * a - 2.0 * a) * (m * m)
    inv = lax.rsqrt(var + _EPS)
    iota = lax.broadcasted_iota(jnp.int32, (1, _G), 1)
    onehot = (bt == iota).astype(jnp.float32)
    rm = jnp.dot(onehot, m, preferred_element_type=jnp.float32)
    rinv = jnp.dot(onehot, inv, preferred_element_type=jnp.float32)
    out = (z - a * rm) * rinv
    return jnp.maximum(w_ref[...] * out + bb_ref[...], 0.0), onehot


def _tc_layer_post(z, s1, s2, cnt, batch2d, dinv_col, w2d, bb2d, a2d):
    n, d = z.shape
    grid = (n // _ROWBLK,)

    def body(z_ref, s1_ref, s2_ref, cnt_ref, bt_ref, dv_ref,
             w_ref, bb_ref, a_ref, out_ref):
        h, _ = _gnorm_rows(z_ref[...], bt_ref[...], s1_ref, s2_ref,
                           cnt_ref, w_ref, bb_ref, a_ref)
        out_ref[...] = dv_ref[...] * h

    return pl.pallas_call(
        body,
        grid=grid,
        in_specs=[
            pl.BlockSpec((_ROWBLK, d), lambda i: (i, 0)),
            pl.BlockSpec((_G, d), lambda i: (0, 0)),
            pl.BlockSpec((_G, d), lambda i: (0, 0)),
            pl.BlockSpec((_G, 128), lambda i: (0, 0)),
            pl.BlockSpec((_ROWBLK, 1), lambda i: (i, 0)),
            pl.BlockSpec((_ROWBLK, 1), lambda i: (i, 0)),
            pl.BlockSpec((1, d), lambda i: (0, 0)),
            pl.BlockSpec((1, d), lambda i: (0, 0)),
            pl.BlockSpec((1, d), lambda i: (0, 0)),
        ],
        out_shape=jax.ShapeDtypeStruct((n, d), jnp.float32),
        out_specs=pl.BlockSpec((_ROWBLK, d), lambda i: (i, 0)),
    )(z, s1, s2, cnt, batch2d, dinv_col, w2d, bb2d, a2d)


def _tc_final_pool(z, s1, s2, cnt, batch2d, w2d, bb2d, a2d):
    n, d = z.shape
    grid = (n // _ROWBLK,)
    last = n // _ROWBLK - 1

    def body(z_ref, s1_ref, s2_ref, cnt_ref, bt_ref,
             w_ref, bb_ref, a_ref, mean_ref, max_ref):
        i = pl.program_id(0)
        h, onehot = _gnorm_rows(z_ref[...], bt_ref[...], s1_ref, s2_ref,
                                cnt_ref, w_ref, bb_ref, a_ref)
        dn = (((0,), (0,)), ((), ()))
        hs = lax.dot_general(onehot, h, dn,
                             preferred_element_type=jnp.float32)
        bt = bt_ref[...]
        gmax = []
        for g in range(_G):
            mask = bt == g
            gmax.append(jnp.max(jnp.where(mask, h, -jnp.inf), axis=0))
        gm = jnp.stack(gmax, axis=0)

        @pl.when(i == 0)
        def _():
            mean_ref[...] = hs
            max_ref[...] = gm

        @pl.when(i > 0)
        def _():
            mean_ref[...] += hs
            max_ref[...] = jnp.maximum(max_ref[...], gm)

        @pl.when(i == last)
        def _():
            mean_ref[...] = mean_ref[...] / cnt_ref[:, 0:1]

    return pl.pallas_call(
        body,
        grid=grid,
        in_specs=[
            pl.BlockSpec((_ROWBLK, d), lambda i: (i, 0)),
            pl.BlockSpec((_G, d), lambda i: (0, 0)),
            pl.BlockSpec((_G, d), lambda i: (0, 0)),
            pl.BlockSpec((_G, 128), lambda i: (0, 0)),
            pl.BlockSpec((_ROWBLK, 1), lambda i: (i, 0)),
            pl.BlockSpec((1, d), lambda i: (0, 0)),
            pl.BlockSpec((1, d), lambda i: (0, 0)),
            pl.BlockSpec((1, d), lambda i: (0, 0)),
        ],
        out_shape=(
            jax.ShapeDtypeStruct((_G, d), jnp.float32),
            jax.ShapeDtypeStruct((_G, d), jnp.float32),
        ),
        out_specs=(
            pl.BlockSpec((_G, d), lambda i: (0, 0)),
            pl.BlockSpec((_G, d), lambda i: (0, 0)),
        ),
    )(z, s1, s2, cnt, batch2d, w2d, bb2d, a2d)


def kernel(x, edge_index, batch, W_in, b_in, W0, b0, gnw0, gnb0, gna0,
           W1, b1, gnw1, gnb1, gna1, W2, b2, gnw2, gnb2, gna2):
    n = x.shape[0]
    n_edges = edge_index.shape[1]
    batch2d = batch[:, None]

    n_pad = ((n + _NS * 16 - 1) // (_NS * 16)) * (_NS * 16)
    cpw = _NBUF * _NW * _CHUNK
    n_e_pad = ((n_edges + cpw - 1) // cpw) * cpw
    n_extra = n_e_pad - n_edges
    pad_src = (jnp.arange(n_extra, dtype=jnp.int32) * 7) % n
    pad_dst = n + (jnp.arange(n_extra, dtype=jnp.int32) % (n_pad - n))
    src = jnp.concatenate([edge_index[0], pad_src])
    dst = jnp.concatenate([edge_index[1], pad_dst])
    packed = jnp.stack([src, dst]).reshape(2, n_e_pad // _CHUNK,
                                           _CHUNK).transpose(1, 0, 2)

    deg_partials = _sc_degree(packed, n_pad).reshape(_NC, n_pad)
    dinv_row, cnt = _tc_prep(deg_partials, batch2d)
    dinv_col = dinv_row[0, :n, None]

    p = _tc_in_proj(x, W_in, b_in[None, :], dinv_col)
    layers = [
        (W0, b0, gnw0, gnb0, gna0),
        (W1, b1, gnw1, gnb1, gna1),
        (W2, b2, gnw2, gnb2, gna2),
    ]
    for li, (w, b, gw, gb, ga) in enumerate(layers):
        qab = _sc_msgpass(p, packed, n_pad)
        z, s1, s2 = _tc_layer_pre(qab, p, dinv_col, w, b[None, :], batch2d)
        if li < 2:
            p = _tc_layer_post(z, s1, s2, cnt, batch2d, dinv_col,
                               gw[None, :], gb[None, :], ga[None, :])
        else:
            pmean, pmax = _tc_final_pool(z, s1, s2, cnt, batch2d,
                                         gw[None, :], gb[None, :],
                                         ga[None, :])
    return jnp.concatenate([pmean, pmax], axis=1)


# speedup vs baseline: 1.1383x; 1.0013x over previous
import functools

import jax
import jax.numpy as jnp
from jax import lax
from jax.experimental import pallas as pl
from jax.experimental.pallas import tpu as pltpu
from jax.experimental.pallas import tpu_sc as plsc

_EPS = 1e-5
_G = 8
_NC = 2
_NS = 16
_LANES = 16
_NW = _NC * _NS
_CHUNK = 128
_NBUF = 2
_ROWBLK = 1000


def _sc_degree(packed, n_pad):
    n_chunks = packed.shape[0]
    nk = n_chunks // _NW
    rpt = n_pad // _NS
    zeros_hbm = jnp.zeros((rpt,), jnp.float32)
    mesh = plsc.VectorSubcoreMesh(core_axis_name="c", subcore_axis_name="s")

    @functools.partial(
        pl.kernel, mesh=mesh,
        out_type=jax.ShapeDtypeStruct((_NC * n_pad,), jnp.float32),
        scratch_types=[
            pltpu.VMEM((2, _CHUNK), jnp.int32),
            pltpu.VMEM((2, _CHUNK), jnp.int32),
            pltpu.VMEM((_CHUNK,), jnp.float32),
            pltpu.VMEM_SHARED((n_pad,), jnp.float32),
        ],
    )
    def deg_kernel(pk_hbm, z_hbm, out_hbm, ib0, ib1, ones_v, acc):
        c = lax.axis_index("c")
        s = lax.axis_index("s")
        wid = s * _NC + c
        ibufs = (ib0, ib1)
        pltpu.sync_copy(z_hbm, acc.at[pl.ds(s * rpt, rpt)])
        for i in range(_CHUNK // _LANES):
            ones_v[pl.ds(i * _LANES, _LANES)] = jnp.ones((_LANES,), jnp.float32)
        plsc.subcore_barrier()

        def scoped(ss0, ss1):
            sss = (ss0, ss1)
            for t in (0, 1):
                pltpu.sync_copy(pk_hbm.at[wid + t * _NW], ibufs[t])

            def body(j, carry):
                for t in (0, 1):
                    k = 2 * j + t
                    pltpu.async_copy(ones_v, acc.at[ibufs[t].at[1]],
                                     sss[t], add=True)

                    @pl.when(k + 2 < nk)
                    def _():
                        pltpu.make_async_copy(
                            ones_v, acc.at[ibufs[t].at[1]], sss[t]).wait()
                        pltpu.sync_copy(pk_hbm.at[wid + (k + 2) * _NW],
                                        ibufs[t])
                return carry

            lax.fori_loop(0, nk // 2, body, 0)
            for t in (0, 1):
                pltpu.make_async_copy(
                    ones_v, acc.at[ibufs[t].at[1]], sss[t]).wait()

        pl.run_scoped(scoped, pltpu.SemaphoreType.DMA,
                      pltpu.SemaphoreType.DMA)
        plsc.subcore_barrier()
        pltpu.sync_copy(acc.at[pl.ds(s * rpt, rpt)],
                        out_hbm.at[pl.ds(c * n_pad + s * rpt, rpt)])

    return deg_kernel(packed, zeros_hbm)


def _sc_msgpass(p, packed, n_pad):
    n, d = p.shape
    n_chunks = packed.shape[0]
    nk = n_chunks // _NW
    rpt = n_pad // _NS
    zeros_hbm = jnp.zeros((rpt, d), jnp.float32)
    mesh = plsc.VectorSubcoreMesh(core_axis_name="c", subcore_axis_name="s")

    nbuf = _NBUF

    @functools.partial(
        pl.kernel, mesh=mesh,
        out_type=jax.ShapeDtypeStruct((_NC, n_pad, d), jnp.float32),
        scratch_types=(
            [pltpu.VMEM((2, _CHUNK), jnp.int32) for _ in range(nbuf)]
            + [pltpu.VMEM((_CHUNK, d), jnp.float32) for _ in range(nbuf)]
            + [pltpu.VMEM_SHARED((n_pad, d), jnp.float32)]
        ),
    )
    def mp_kernel(p_hbm, pk_hbm, z_hbm, out_hbm, *bufs):
        ibufs = bufs[:nbuf]
        rbufs = bufs[nbuf:2 * nbuf]
        acc = bufs[2 * nbuf]
        c = lax.axis_index("c")
        s = lax.axis_index("s")
        wid = s * _NC + c
        pltpu.sync_copy(z_hbm, acc.at[pl.ds(s * rpt, rpt)])
        plsc.subcore_barrier()

        def scoped(*sems):
            sgs = sems[:nbuf]
            sss = sems[nbuf:]
            for t in range(nbuf):
                pltpu.sync_copy(pk_hbm.at[wid + t * _NW], ibufs[t])
                pltpu.async_copy(p_hbm.at[ibufs[t].at[0]], rbufs[t], sgs[t])

            def body(j, carry):
                for t in range(nbuf):
                    k = nbuf * j + t
                    pltpu.make_async_copy(
                        p_hbm.at[ibufs[t].at[0]], rbufs[t], sgs[t]).wait()
                    pltpu.async_copy(rbufs[t], acc.at[ibufs[t].at[1]],
                                     sss[t], add=True)

                    @pl.when(k + nbuf < nk)
                    def _():
                        pltpu.make_async_copy(
                            rbufs[t], acc.at[ibufs[t].at[1]], sss[t]).wait()
                        pltpu.sync_copy(pk_hbm.at[wid + (k + nbuf) * _NW],
                                        ibufs[t])
                        pltpu.async_copy(p_hbm.at[ibufs[t].at[0]],
                                         rbufs[t], sgs[t])
                return carry

            lax.fori_loop(0, nk // nbuf, body, 0)
            for t in range(nbuf):
                pltpu.make_async_copy(
                    rbufs[t], acc.at[ibufs[t].at[1]], sss[t]).wait()

        pl.run_scoped(scoped,
                      *([pltpu.SemaphoreType.DMA] * (2 * nbuf)))
        plsc.subcore_barrier()
        pltpu.sync_copy(acc.at[pl.ds(s * rpt, rpt)],
                        out_hbm.at[c, pl.ds(s * rpt, rpt)])

    return mp_kernel(p, packed, zeros_hbm)


def _tc_prep(deg_partials, batch2d):
    n_pad = deg_partials.shape[1]

    def body(dp_ref, b_ref, dinv_ref, cnt_ref):
        deg = dp_ref[0, :] + dp_ref[1, :] + 1.0
        dinv_ref[...] = lax.rsqrt(deg)[None, :]
        iota = lax.broadcasted_iota(jnp.int32, (1, _G), 1)
        onehot = (b_ref[...] == iota).astype(jnp.float32)
        cnt = jnp.maximum(jnp.sum(onehot, axis=0), 1.0)
        cnt_ref[...] = jnp.broadcast_to(cnt[:, None], (_G, 128))

    return pl.pallas_call(
        body,
        out_shape=(
            jax.ShapeDtypeStruct((1, n_pad), jnp.float32),
            jax.ShapeDtypeStruct((_G, 128), jnp.float32),
        ),
    )(deg_partials, batch2d)


def _tc_in_proj(x, w, b2d, dinv_col):
    n, d_in = x.shape
    d_out = w.shape[1]
    grid = (n // _ROWBLK,)

    def body(x_ref, w_ref, b_ref, dv_ref, out_ref):
        h = jnp.dot(x_ref[...], w_ref[...],
                    preferred_element_type=jnp.float32) + b_ref[...]
        out_ref[...] = dv_ref[...] * h

    return pl.pallas_call(
        body,
        grid=grid,
        in_specs=[
            pl.BlockSpec((_ROWBLK, d_in), lambda i: (i, 0)),
            pl.BlockSpec((d_in, d_out), lambda i: (0, 0)),
            pl.BlockSpec((1, d_out), lambda i: (0, 0)),
            pl.BlockSpec((_ROWBLK, 1), lambda i: (i, 0)),
        ],
        out_shape=jax.ShapeDtypeStruct((n, d_out), jnp.float32),
        out_specs=pl.BlockSpec((_ROWBLK, d_out), lambda i: (i, 0)),
    )(x, w, b2d, dinv_col)


def _tc_layer_pre(qab, p, dinv_col, w, b2d, batch2d):
    n, d_in = p.shape
    d_out = w.shape[1]
    grid = (n // _ROWBLK,)

    def body(q_ref, p_ref, dv_ref, w_ref, b_ref, bt_ref,
             z_ref, s1_ref, s2_ref):
        i = pl.program_id(0)
        agg = dv_ref[...] * (q_ref[0] + q_ref[1] + p_ref[...])
        z = jnp.dot(agg, w_ref[...],
                    preferred_element_type=jnp.float32) + b_ref[...]
        z_ref[...] = z
        iota = lax.broadcasted_iota(jnp.int32, (1, _G), 1)
        onehot = (bt_ref[...] == iota).astype(jnp.float32)
        dn = (((0,), (0,)), ((), ()))
        s1 = lax.dot_general(onehot, z, dn,
                             preferred_element_type=jnp.float32)
        s2 = lax.dot_general(onehot, z * z, dn,
                             preferred_element_type=jnp.float32)

        @pl.when(i == 0)
        def _():
            s1_ref[...] = s1
            s2_ref[...] = s2

        @pl.when(i > 0)
        def _():
            s1_ref[...] += s1
            s2_ref[...] += s2

    return pl.pallas_call(
        body,
        grid=grid,
        in_specs=[
            pl.BlockSpec((2, _ROWBLK, d_in), lambda i: (0, i, 0)),
            pl.BlockSpec((_ROWBLK, d_in), lambda i: (i, 0)),
            pl.BlockSpec((_ROWBLK, 1), lambda i: (i, 0)),
            pl.BlockSpec((d_in, d_out), lambda i: (0, 0)),
            pl.BlockSpec((1, d_out), lambda i: (0, 0)),
            pl.BlockSpec((_ROWBLK, 1), lambda i: (i, 0)),
        ],
        out_shape=(
            jax.ShapeDtypeStruct((n, d_out), jnp.float32),
            jax.ShapeDtypeStruct((_G, d_out), jnp.float32),
            jax.ShapeDtypeStruct((_G, d_out), jnp.float32),
        ),
        out_specs=(
            pl.BlockSpec((_ROWBLK, d_out), lambda i: (i, 0)),
            pl.BlockSpec((_G, d_out), lambda i: (0, 0)),
            pl.BlockSpec((_G, d_out), lambda i: (0, 0)),
        ),
    )(qab, p, dinv_col, w, b2d, batch2d)


def _gnorm_rows(z, bt, s1_ref, s2_ref, cnt_ref, w_ref, bb_ref, a_ref):
    cntc = cnt_ref[:, 0:1]
    m = s1_ref[...] / cntc
    e2 = s2_ref[...] / cntc
    a = a_ref[...]
    var = e2 + (a * a - 2.0 * a) * (m * m)
    inv = lax.rsqrt(var + _EPS)
    iota = lax.broadcasted_iota(jnp.int32, (1, _G), 1)
    onehot = (bt == iota).astype(jnp.float32)
    rm = jnp.dot(onehot, m, preferred_element_type=jnp.float32)
    rinv = jnp.dot(onehot, inv, preferred_element_type=jnp.float32)
    out = (z - a * rm) * rinv
    return jnp.maximum(w_ref[...] * out + bb_ref[...], 0.0), onehot


def _tc_layer_post(z, s1, s2, cnt, batch2d, dinv_col, w2d, bb2d, a2d):
    n, d = z.shape
    grid = (n // _ROWBLK,)

    def body(z_ref, s1_ref, s2_ref, cnt_ref, bt_ref, dv_ref,
             w_ref, bb_ref, a_ref, out_ref):
        h, _ = _gnorm_rows(z_ref[...], bt_ref[...], s1_ref, s2_ref,
                           cnt_ref, w_ref, bb_ref, a_ref)
        out_ref[...] = dv_ref[...] * h

    return pl.pallas_call(
        body,
        grid=grid,
        in_specs=[
            pl.BlockSpec((_ROWBLK, d), lambda i: (i, 0)),
            pl.BlockSpec((_G, d), lambda i: (0, 0)),
            pl.BlockSpec((_G, d), lambda i: (0, 0)),
            pl.BlockSpec((_G, 128), lambda i: (0, 0)),
            pl.BlockSpec((_ROWBLK, 1), lambda i: (i, 0)),
            pl.BlockSpec((_ROWBLK, 1), lambda i: (i, 0)),
            pl.BlockSpec((1, d), lambda i: (0, 0)),
            pl.BlockSpec((1, d), lambda i: (0, 0)),
            pl.BlockSpec((1, d), lambda i: (0, 0)),
        ],
        out_shape=jax.ShapeDtypeStruct((n, d), jnp.float32),
        out_specs=pl.BlockSpec((_ROWBLK, d), lambda i: (i, 0)),
    )(z, s1, s2, cnt, batch2d, dinv_col, w2d, bb2d, a2d)


def _tc_final_pool(z, s1, s2, cnt, batch2d, w2d, bb2d, a2d):
    n, d = z.shape
    grid = (n // _ROWBLK,)
    last = n // _ROWBLK - 1

    def body(z_ref, s1_ref, s2_ref, cnt_ref, bt_ref,
             w_ref, bb_ref, a_ref, mean_ref, max_ref):
        i = pl.program_id(0)
        h, onehot = _gnorm_rows(z_ref[...], bt_ref[...], s1_ref, s2_ref,
                                cnt_ref, w_ref, bb_ref, a_ref)
        dn = (((0,), (0,)), ((), ()))
        hs = lax.dot_general(onehot, h, dn,
                             preferred_element_type=jnp.float32)
        bt = bt_ref[...]
        gmax = []
        for g in range(_G):
            mask = bt == g
            gmax.append(jnp.max(jnp.where(mask, h, -jnp.inf), axis=0))
        gm = jnp.stack(gmax, axis=0)

        @pl.when(i == 0)
        def _():
            mean_ref[...] = hs
            max_ref[...] = gm

        @pl.when(i > 0)
        def _():
            mean_ref[...] += hs
            max_ref[...] = jnp.maximum(max_ref[...], gm)

        @pl.when(i == last)
        def _():
            mean_ref[...] = mean_ref[...] / cnt_ref[:, 0:1]

    return pl.pallas_call(
        body,
        grid=grid,
        in_specs=[
            pl.BlockSpec((_ROWBLK, d), lambda i: (i, 0)),
            pl.BlockSpec((_G, d), lambda i: (0, 0)),
            pl.BlockSpec((_G, d), lambda i: (0, 0)),
            pl.BlockSpec((_G, 128), lambda i: (0, 0)),
            pl.BlockSpec((_ROWBLK, 1), lambda i: (i, 0)),
            pl.BlockSpec((1, d), lambda i: (0, 0)),
            pl.BlockSpec((1, d), lambda i: (0, 0)),
            pl.BlockSpec((1, d), lambda i: (0, 0)),
        ],
        out_shape=(
            jax.ShapeDtypeStruct((_G, d), jnp.float32),
            jax.ShapeDtypeStruct((_G, d), jnp.float32),
        ),
        out_specs=(
            pl.BlockSpec((_G, d), lambda i: (0, 0)),
            pl.BlockSpec((_G, d), lambda i: (0, 0)),
        ),
    )(z, s1, s2, cnt, batch2d, w2d, bb2d, a2d)


def kernel(x, edge_index, batch, W_in, b_in, W0, b0, gnw0, gnb0, gna0,
           W1, b1, gnw1, gnb1, gna1, W2, b2, gnw2, gnb2, gna2):
    n = x.shape[0]
    n_edges = edge_index.shape[1]
    batch2d = batch[:, None]

    n_pad = ((n + _NS * 16 - 1) // (_NS * 16)) * (_NS * 16)
    cpw = _NBUF * _NW * _CHUNK
    n_e_pad = ((n_edges + cpw - 1) // cpw) * cpw
    n_extra = n_e_pad - n_edges
    pad_src = (jnp.arange(n_extra, dtype=jnp.int32) * 7) % n
    pad_dst = n + (jnp.arange(n_extra, dtype=jnp.int32) % (n_pad - n))
    src = jnp.concatenate([edge_index[0], pad_src])
    dst = jnp.concatenate([edge_index[1], pad_dst])
    packed = jnp.stack([src, dst]).reshape(2, n_e_pad // _CHUNK,
                                           _CHUNK).transpose(1, 0, 2)

    deg_partials = _sc_degree(packed, n_pad).reshape(_NC, n_pad)
    dinv_row, cnt = _tc_prep(deg_partials, batch2d)
    dinv_col = dinv_row[0, :n, None]

    p = _tc_in_proj(x, W_in, b_in[None, :], dinv_col)
    layers = [
        (W0, b0, gnw0, gnb0, gna0),
        (W1, b1, gnw1, gnb1, gna1),
        (W2, b2, gnw2, gnb2, gna2),
    ]
    for li, (w, b, gw, gb, ga) in enumerate(layers):
        qab = _sc_msgpass(p, packed, n_pad)
        z, s1, s2 = _tc_layer_pre(qab, p, dinv_col, w, b[None, :], batch2d)
        if li < 2:
            p = _tc_layer_post(z, s1, s2, cnt, batch2d, dinv_col,
                               gw[None, :], gb[None, :], ga[None, :])
        else:
            pmean, pmax = _tc_final_pool(z, s1, s2, cnt, batch2d,
                                         gw[None, :], gb[None, :],
                                         ga[None, :])
    return jnp.concatenate([pmean, pmax], axis=1)
